# Initial kernel scaffold; baseline (speedup 1.0000x reference)
import jax
import jax.numpy as jnp
from jax.experimental import pallas as pl


def kernel(batch, emb_table):
    raise NotImplementedError("write your pallas kernel here")


# speedup vs baseline: 4.6072x; 4.6072x over previous
import jax
import jax.numpy as jnp
from jax.experimental import pallas as pl


def _posemb_kernel(batch_ref, tab_ref, out_ref):
    mask = batch_ref[...] != 0
    tab = tab_ref[...]
    out_ref[...] = jnp.where(mask, tab, 0.0)


def kernel(batch, emb_table):
    B, L = batch.shape
    E = emb_table.shape[1]
    tab = emb_table[1:L + 1].reshape(1, L, E)
    batch3 = batch.reshape(B, L, 1)
    BT = 64
    grid = (B // BT,)
    return pl.pallas_call(
        _posemb_kernel,
        grid=grid,
        in_specs=[
            pl.BlockSpec((BT, L, 1), lambda i: (i, 0, 0)),
            pl.BlockSpec((1, L, E), lambda i: (0, 0, 0)),
        ],
        out_specs=pl.BlockSpec((BT, L, E), lambda i: (i, 0, 0)),
        out_shape=jax.ShapeDtypeStruct((B, L, E), jnp.float32),
    )(batch3, tab)


# speedup vs baseline: 7.4312x; 1.6130x over previous
import jax
import jax.numpy as jnp
from jax.experimental import pallas as pl


def _posemb_kernel(batch_ref, tab_ref, out_ref):
    bt, l, e = out_ref.shape
    mask = batch_ref[...] != 0
    mask_r = jax.lax.broadcast_in_dim(mask, (bt, 1, l), (0, 2))
    mask3 = jnp.swapaxes(mask_r, 1, 2)
    tab = tab_ref[...]
    out_ref[...] = jnp.where(mask3, tab, 0.0)


def kernel(batch, emb_table):
    B, L = batch.shape
    E = emb_table.shape[1]
    tab = emb_table[1:L + 1].reshape(1, L, E)
    BT = 64
    grid = (B // BT,)
    return pl.pallas_call(
        _posemb_kernel,
        grid=grid,
        in_specs=[
            pl.BlockSpec((BT, L), lambda i: (i, 0)),
            pl.BlockSpec((1, L, E), lambda i: (0, 0, 0)),
        ],
        out_specs=pl.BlockSpec((BT, L, E), lambda i: (i, 0, 0)),
        out_shape=jax.ShapeDtypeStruct((B, L, E), jnp.float32),
    )(batch, tab)


# speedup vs baseline: 7.4792x; 1.0065x over previous
import jax
import jax.numpy as jnp
from jax.experimental import pallas as pl


def _posemb_kernel(batch_ref, tab_ref, out_ref):
    bt, l, e = out_ref.shape
    mask = batch_ref[...] != 0
    mask_r = jax.lax.broadcast_in_dim(mask, (bt, 1, l), (0, 2))
    mask3 = jnp.swapaxes(mask_r, 1, 2)
    tab = tab_ref[...]
    out_ref[...] = jnp.where(mask3, tab, 0.0)


def kernel(batch, emb_table):
    B, L = batch.shape
    E = emb_table.shape[1]
    tab = emb_table[1:L + 1].reshape(1, L, E)
    BT = 128
    grid = (B // BT,)
    return pl.pallas_call(
        _posemb_kernel,
        grid=grid,
        in_specs=[
            pl.BlockSpec((BT, L), lambda i: (i, 0)),
            pl.BlockSpec((1, L, E), lambda i: (0, 0, 0)),
        ],
        out_specs=pl.BlockSpec((BT, L, E), lambda i: (i, 0, 0)),
        out_shape=jax.ShapeDtypeStruct((B, L, E), jnp.float32),
    )(batch, tab)
